# baseline (speedup 1.0000x reference)
import functools

import jax
import jax.numpy as jnp
from jax import lax
from jax.experimental import pallas as pl
from jax.experimental.pallas import tpu as pltpu
from jax.experimental.pallas import tpu_sc as plsc

N = 5000
K = 17
GR = 4
GI = 72
D = 512
RB = 200
MB = 1000
NC, NS = 2, 16
NW = NC * NS
NPAD = 5120
RPW = NPAD // NW
GPW = RPW // GR
BIGI = 2 ** 30


def _topk_body(ct_ref, cb_ref, out_ref):
    cxi = cb_ref[:, 0:1]
    cyi = cb_ref[:, 1:2]
    cxj = ct_ref[0:1, :]
    cyj = ct_ref[1:2, :]
    dx = cxj - cxi
    dy = cyj - cyi
    d = jnp.sqrt(dx * dx + dy * dy)
    col = lax.broadcasted_iota(jnp.int32, (RB, N), 1)
    kcol = lax.broadcasted_iota(jnp.int32, (RB, 128), 1)
    acc = jnp.zeros((RB, 128), jnp.int32)
    for k in range(K):
        m = jnp.min(d, axis=1, keepdims=True)
        cand = jnp.where(d == m, col, BIGI)
        a = jnp.min(cand, axis=1, keepdims=True)
        acc = jnp.where(kcol == k, a * K + k, acc)
        d = jnp.where(col == a, jnp.inf, d)
    out_ref[...] = acc


def _matmul_body(x_ref, w_ref, o_ref):
    y = jnp.dot(x_ref[...], w_ref[...], preferred_element_type=jnp.float32)
    lo = lax.bitcast_convert_type(y[:, :D // 2].astype(jnp.bfloat16),
                                  jnp.uint16).astype(jnp.int32)
    hi = lax.bitcast_convert_type(y[:, D // 2:].astype(jnp.bfloat16),
                                  jnp.int16).astype(jnp.int32) << 16
    o_ref[...] = hi | lo


def _tree_sum(vals):
    while len(vals) > 1:
        nxt = [vals[i] + vals[i + 1] for i in range(0, len(vals) - 1, 2)]
        if len(vals) % 2:
            nxt.append(vals[-1])
        vals = nxt
    return vals[0]


def _sc_gather_sum(p_hbm, idx_hbm, b_hbm, out_hbm, idx_v, rows_v0, rows_v1,
                   rows_v2, rows_v3, b_v, out_buf, sem0, sem1, sem2, sem3):
    wid = lax.axis_index("s") * NC + lax.axis_index("c")
    baser = wid * RPW
    baseg = wid * GPW
    pltpu.sync_copy(idx_hbm.at[pl.ds(baseg, GPW)], idx_v)
    pltpu.sync_copy(b_hbm, b_v)
    bufs = (rows_v0, rows_v1, rows_v2, rows_v3)
    sems = (sem0, sem1, sem2, sem3)

    for g0 in range(3):
        pltpu.async_copy(p_hbm.at[idx_v.at[g0]], bufs[g0], sems[g0])

    def qbody(q, carry):
        for b4 in range(4):
            g = q * 4 + b4
            cur, nxt = b4, (b4 + 3) % 4

            @pl.when(g + 3 < GPW)
            def _():
                pltpu.async_copy(p_hbm.at[idx_v.at[g + 3]], bufs[nxt],
                                 sems[nxt])

            pltpu.make_async_copy(p_hbm.at[idx_v.at[g]], bufs[cur],
                                  sems[cur]).wait()
            rows = bufs[cur]

            def cbody(c, carry2, rows=rows, b4=b4):
                slo = pl.ds(c * 32, 16)
                shi = pl.ds(c * 32 + 16, 16)
                blo = b_v[slo]
                bhi = b_v[shi]
                for rr in range(GR):
                    parts = []
                    for j in range(K):
                        w = rows[rr * K + j, pl.ds(c * 16, 16)]
                        parts.append(
                            (lax.bitcast_convert_type(w << 16, jnp.float32),
                             lax.bitcast_convert_type(w & -65536,
                                                      jnp.float32)))
                    lo = _tree_sum([p[0] for p in parts] + [blo])
                    hi = _tree_sum([p[1] for p in parts] + [bhi])
                    out_buf[b4 * GR + rr, slo] = jnp.maximum(lo, 0.0)
                    out_buf[b4 * GR + rr, shi] = jnp.maximum(hi, 0.0)
                return carry2

            lax.fori_loop(0, D // 32, cbody, 0)

        start = baser + q * 16

        @pl.when(start + 16 <= N)
        def _():
            pltpu.sync_copy(out_buf, out_hbm.at[pl.ds(start, 16)])

        @pl.when((start < N) & (start + 16 > N))
        def _():
            pltpu.sync_copy(out_buf.at[pl.ds(0, 8)],
                            out_hbm.at[pl.ds(start, 8)])

        return carry

    lax.fori_loop(0, GPW // 4, qbody, 0)


def kernel(features, rois, W, b):
    centers = rois.mean(axis=1)
    ct = centers.T
    v2 = W.reshape(D, D, K).transpose(1, 2, 0).reshape(D, K * D)
    v2 = (v2.reshape(D, K, 16, 2, 16).transpose(0, 1, 3, 2, 4)
          .reshape(D, K * D))

    gidx = pl.pallas_call(
        _topk_body,
        grid=(N // RB,),
        in_specs=[
            pl.BlockSpec((2, N), lambda i: (0, 0)),
            pl.BlockSpec((RB, 2), lambda i: (i, 0)),
        ],
        out_specs=pl.BlockSpec((RB, 128), lambda i: (i, 0)),
        out_shape=jax.ShapeDtypeStruct((N, 128), jnp.int32),
    )(ct, centers)

    p = pl.pallas_call(
        _matmul_body,
        grid=(N // MB, K * D // D),
        in_specs=[
            pl.BlockSpec((MB, D), lambda i, j: (i, 0)),
            pl.BlockSpec((D, D), lambda i, j: (0, j)),
        ],
        out_specs=pl.BlockSpec((MB, D // 2), lambda i, j: (i, j)),
        out_shape=jax.ShapeDtypeStruct((N, K * D // 2), jnp.int32),
    )(features, v2)
    p85 = p.reshape(N * K, D // 2)

    idx_pad = jnp.pad(gidx[:, :K], ((0, NPAD - N), (0, 0)))
    idx_grp = jnp.pad(idx_pad.reshape(NPAD // GR, GR * K),
                      ((0, 0), (0, GI - GR * K)))

    mesh = plsc.VectorSubcoreMesh(core_axis_name="c", subcore_axis_name="s")
    sc_call = functools.partial(
        pl.kernel,
        mesh=mesh,
        out_type=jax.ShapeDtypeStruct((N, D), jnp.float32),
        scratch_types=[
            pltpu.VMEM((GPW, GI), jnp.int32),
            pltpu.VMEM((GI, D // 2), jnp.int32),
            pltpu.VMEM((GI, D // 2), jnp.int32),
            pltpu.VMEM((GI, D // 2), jnp.int32),
            pltpu.VMEM((GI, D // 2), jnp.int32),
            pltpu.VMEM((D,), jnp.float32),
            pltpu.VMEM((16, D), jnp.float32),
            pltpu.SemaphoreType.DMA,
            pltpu.SemaphoreType.DMA,
            pltpu.SemaphoreType.DMA,
            pltpu.SemaphoreType.DMA,
        ],
    )(_sc_gather_sum)
    return sc_call(p85, idx_grp, b)


# speedup vs baseline: 1.0081x; 1.0081x over previous
import functools

import jax
import jax.numpy as jnp
from jax import lax
from jax.experimental import pallas as pl
from jax.experimental.pallas import tpu as pltpu
from jax.experimental.pallas import tpu_sc as plsc

N = 5000
K = 17
GR = 4
GI = 72
D = 512
RB = 200
MB = 1000
NC, NS = 2, 16
NW = NC * NS
NPAD = 5120
RPW = NPAD // NW
GPW = RPW // GR
BIGI = 2 ** 30


def _fused_body(ct_ref, cb_ref, x_ref, w_ref, gidx_ref, p_ref):
    cxi = cb_ref[:, 0:1]
    cyi = cb_ref[:, 1:2]
    cxj = ct_ref[0:1, :]
    cyj = ct_ref[1:2, :]
    dx = cxj - cxi
    dy = cyj - cyi
    d = jnp.sqrt(dx * dx + dy * dy)
    col = lax.broadcasted_iota(jnp.int32, (RB, N), 1)
    kcol = lax.broadcasted_iota(jnp.int32, (RB, 128), 1)
    acc = jnp.zeros((RB, 128), jnp.int32)
    for k in range(K):
        m = jnp.min(d, axis=1, keepdims=True)
        cand = jnp.where(d == m, col, BIGI)
        a = jnp.min(cand, axis=1, keepdims=True)
        acc = jnp.where(kcol == k, a * K + k, acc)
        d = jnp.where(col == a, jnp.inf, d)
    gidx_ref[...] = acc

    x = x_ref[...]
    for j in range(K):
        y = jnp.dot(x, w_ref[:, j * D:(j + 1) * D],
                    preferred_element_type=jnp.float32)
        lo = lax.bitcast_convert_type(y[:, :D // 2].astype(jnp.bfloat16),
                                      jnp.uint16).astype(jnp.int32)
        hi = lax.bitcast_convert_type(y[:, D // 2:].astype(jnp.bfloat16),
                                      jnp.int16).astype(jnp.int32) << 16
        p_ref[:, j * (D // 2):(j + 1) * (D // 2)] = hi | lo


def _tree_sum(vals):
    while len(vals) > 1:
        nxt = [vals[i] + vals[i + 1] for i in range(0, len(vals) - 1, 2)]
        if len(vals) % 2:
            nxt.append(vals[-1])
        vals = nxt
    return vals[0]


def _sc_gather_sum(p_hbm, idx_hbm, b_hbm, out_hbm, idx_v, rows_v0, rows_v1,
                   rows_v2, rows_v3, b_v, out_buf, sem0, sem1, sem2, sem3):
    wid = lax.axis_index("s") * NC + lax.axis_index("c")
    baser = wid * RPW
    baseg = wid * GPW
    pltpu.sync_copy(idx_hbm.at[pl.ds(baseg, GPW)], idx_v)
    pltpu.sync_copy(b_hbm, b_v)
    bufs = (rows_v0, rows_v1, rows_v2, rows_v3)
    sems = (sem0, sem1, sem2, sem3)

    for g0 in range(3):
        pltpu.async_copy(p_hbm.at[idx_v.at[g0]], bufs[g0], sems[g0])

    def qbody(q, carry):
        for b4 in range(4):
            g = q * 4 + b4
            cur, nxt = b4, (b4 + 3) % 4

            @pl.when(g + 3 < GPW)
            def _():
                pltpu.async_copy(p_hbm.at[idx_v.at[g + 3]], bufs[nxt],
                                 sems[nxt])

            pltpu.make_async_copy(p_hbm.at[idx_v.at[g]], bufs[cur],
                                  sems[cur]).wait()
            rows = bufs[cur]

            def cbody(c, carry2, rows=rows, b4=b4):
                slo = pl.ds(c * 32, 16)
                shi = pl.ds(c * 32 + 16, 16)
                blo = b_v[slo]
                bhi = b_v[shi]
                for rr in range(GR):
                    parts = []
                    for j in range(K):
                        w = rows[rr * K + j, pl.ds(c * 16, 16)]
                        parts.append(
                            (lax.bitcast_convert_type(w << 16, jnp.float32),
                             lax.bitcast_convert_type(w & -65536,
                                                      jnp.float32)))
                    lo = _tree_sum([p[0] for p in parts] + [blo])
                    hi = _tree_sum([p[1] for p in parts] + [bhi])
                    out_buf[b4 * GR + rr, slo] = jnp.maximum(lo, 0.0)
                    out_buf[b4 * GR + rr, shi] = jnp.maximum(hi, 0.0)
                return carry2

            lax.fori_loop(0, D // 32, cbody, 0)

        start = baser + q * 16

        @pl.when(start + 16 <= N)
        def _():
            pltpu.sync_copy(out_buf, out_hbm.at[pl.ds(start, 16)])

        @pl.when((start < N) & (start + 16 > N))
        def _():
            pltpu.sync_copy(out_buf.at[pl.ds(0, 8)],
                            out_hbm.at[pl.ds(start, 8)])

        return carry

    lax.fori_loop(0, GPW // 4, qbody, 0)


def kernel(features, rois, W, b):
    centers = rois.mean(axis=1)
    ct = centers.T
    v2 = W.reshape(D, D, K).transpose(1, 2, 0).reshape(D, K * D)
    v2 = (v2.reshape(D, K, 16, 2, 16).transpose(0, 1, 3, 2, 4)
          .reshape(D, K * D))

    gidx, p = pl.pallas_call(
        _fused_body,
        grid=(N // RB,),
        in_specs=[
            pl.BlockSpec((2, N), lambda i: (0, 0)),
            pl.BlockSpec((RB, 2), lambda i: (i, 0)),
            pl.BlockSpec((RB, D), lambda i: (i, 0)),
            pl.BlockSpec((D, K * D), lambda i: (0, 0)),
        ],
        out_specs=[
            pl.BlockSpec((RB, 128), lambda i: (i, 0)),
            pl.BlockSpec((RB, K * D // 2), lambda i: (i, 0)),
        ],
        out_shape=[
            jax.ShapeDtypeStruct((N, 128), jnp.int32),
            jax.ShapeDtypeStruct((N, K * D // 2), jnp.int32),
        ],
    )(ct, centers, features, v2)
    p85 = p.reshape(N * K, D // 2)

    idx_pad = jnp.pad(gidx[:, :K], ((0, NPAD - N), (0, 0)))
    idx_grp = jnp.pad(idx_pad.reshape(NPAD // GR, GR * K),
                      ((0, 0), (0, GI - GR * K)))

    mesh = plsc.VectorSubcoreMesh(core_axis_name="c", subcore_axis_name="s")
    sc_call = functools.partial(
        pl.kernel,
        mesh=mesh,
        out_type=jax.ShapeDtypeStruct((N, D), jnp.float32),
        scratch_types=[
            pltpu.VMEM((GPW, GI), jnp.int32),
            pltpu.VMEM((GI, D // 2), jnp.int32),
            pltpu.VMEM((GI, D // 2), jnp.int32),
            pltpu.VMEM((GI, D // 2), jnp.int32),
            pltpu.VMEM((GI, D // 2), jnp.int32),
            pltpu.VMEM((D,), jnp.float32),
            pltpu.VMEM((16, D), jnp.float32),
            pltpu.SemaphoreType.DMA,
            pltpu.SemaphoreType.DMA,
            pltpu.SemaphoreType.DMA,
            pltpu.SemaphoreType.DMA,
        ],
    )(_sc_gather_sum)
    return sc_call(p85, idx_grp, b)
